# Initial kernel scaffold; baseline (speedup 1.0000x reference)
import jax
import jax.numpy as jnp
from jax.experimental import pallas as pl


def kernel(S_batch, channel_idx, G_ema, update_count):
    raise NotImplementedError("write your pallas kernel here")


# speedup vs baseline: 2.7954x; 2.7954x over previous
import functools

import jax
import jax.numpy as jnp
from jax.experimental import pallas as pl
from jax.experimental.pallas import tpu as pltpu

_MOMENTUM = 0.99
_BK = 128


def _mean_combine_kernel(s_ref, p_ref, pt_ref, g_ref, o_ref, acc_ref, *, steps, inv_b):
    step = pl.program_id(0)

    @pl.when(step == 0)
    def _init():
        acc_ref[...] = jnp.zeros_like(acc_ref)

    acc_ref[...] += jnp.sum(s_ref[...], axis=0)

    @pl.when(step == steps - 1)
    def _finish():
        pg = jnp.dot(p_ref[...], g_ref[...], preferred_element_type=jnp.float32)
        sub = jnp.dot(pg, pt_ref[...], preferred_element_type=jnp.float32)
        s_mean = acc_ref[...] * inv_b
        o_ref[...] = _MOMENTUM * sub + (1.0 - _MOMENTUM) * s_mean


def kernel(S_batch, channel_idx, G_ema, update_count):
    B, C, _ = S_batch.shape
    tot = G_ema.shape[0]
    tp = max(128, ((tot + 127) // 128) * 128)

    idx = channel_idx.astype(jnp.int32)
    p = (idx[:, None] == jnp.arange(tp, dtype=jnp.int32)[None, :]).astype(jnp.float32)
    g = jnp.pad(G_ema.astype(jnp.float32), ((0, tp - tot), (0, tp - tot)))

    steps = B // _BK
    out = pl.pallas_call(
        functools.partial(_mean_combine_kernel, steps=steps, inv_b=1.0 / B),
        grid=(steps,),
        in_specs=[
            pl.BlockSpec((_BK, C, C), lambda i: (i, 0, 0)),
            pl.BlockSpec((C, tp), lambda i: (0, 0)),
            pl.BlockSpec((tp, C), lambda i: (0, 0)),
            pl.BlockSpec((tp, tp), lambda i: (0, 0)),
        ],
        out_specs=pl.BlockSpec((C, C), lambda i: (0, 0)),
        out_shape=jax.ShapeDtypeStruct((C, C), jnp.float32),
        scratch_shapes=[pltpu.VMEM((C, C), jnp.float32)],
        compiler_params=pltpu.CompilerParams(
            dimension_semantics=("arbitrary",),
        ),
    )(S_batch.astype(jnp.float32), p, p.T, g)
    return out
